# baseline (speedup 1.0000x reference)
import functools

import jax
import jax.numpy as jnp
from jax import lax
from jax.experimental import pallas as pl
from jax.experimental.pallas import tpu as pltpu
from jax.experimental.pallas import tpu_sc as plsc

_NC = 2
_NS = 16
_NW = _NC * _NS

_LANES = 16

_BAGS_PER_CHUNK = 8
_SEG = 100
_BLKP = 2048


def _repack_body(t_ref, o_ref):
    o_ref[:, 0:64] = t_ref[...].T.astype(jnp.bfloat16)


def _repack(embed_weight):
    num_emb, edim = embed_weight.shape
    t = embed_weight.T
    return pl.pallas_call(
        _repack_body,
        grid=(pl.cdiv(num_emb, _BLKP),),
        in_specs=[pl.BlockSpec((edim, _BLKP), lambda i: (0, i))],
        out_specs=pl.BlockSpec((_BLKP, 2 * edim), lambda i: (i, 0)),
        out_shape=jax.ShapeDtypeStruct((num_emb, 2 * edim), jnp.bfloat16),
    )(t)


@functools.lru_cache(maxsize=None)
def _make_bag_mean(batch, hist, edim, num_emb):
    bags_per_w = batch // _NW
    chunks_per_w = bags_per_w // _BAGS_PER_CHUNK
    idx_per_chunk = _BAGS_PER_CHUNK * hist
    nseg = idx_per_chunk // _SEG
    cvecs = edim // _LANES
    wdim = 2 * edim

    mesh = plsc.VectorSubcoreMesh(core_axis_name="c", subcore_axis_name="s")

    @functools.partial(
        pl.kernel,
        out_type=jax.ShapeDtypeStruct((batch, edim), jnp.float32),
        mesh=mesh,
        scratch_types=[
            pltpu.VMEM((nseg, _SEG), jnp.int32),
            pltpu.VMEM((nseg, _SEG, wdim), jnp.bfloat16),
            pltpu.VMEM((bags_per_w, edim), jnp.float32),
            pltpu.SemaphoreType.DMA,
        ],
        compiler_params=pltpu.CompilerParams(
            use_tc_tiling_on_sc=False, needs_layout_passes=False),
    )
    def bag_mean(idx_hbm, table_hbm, out_hbm, idx_v, rows_v, out_v, sem):
        wid = lax.axis_index("s") * _NC + lax.axis_index("c")
        chunk0 = wid * chunks_per_w

        def chunk_body(g, carry):
            pltpu.sync_copy(idx_hbm.at[chunk0 + g], idx_v)
            for j in range(nseg):
                pltpu.async_copy(table_hbm.at[idx_v.at[j]], rows_v.at[j], sem)
            for j in range(nseg):
                pltpu.make_async_copy(
                    table_hbm.at[idx_v.at[j]], rows_v.at[j], sem).wait()
            for b in range(_BAGS_PER_CHUNK):
                j, r0 = divmod(b * hist, _SEG)
                unroll = 5
                def rbody(r, accs, j=j, r0=r0):
                    a = list(accs)
                    base = r0 + r * unroll
                    for u in range(unroll):
                        for h in range(cvecs // 2):
                            v = rows_v[j, base + u,
                                       pl.ds(h * 2 * _LANES, 2 * _LANES)]
                            even, odd = plsc.unpack(
                                v, format=plsc.PackFormat.INTERLEAVED)
                            a[2 * h] = a[2 * h] + even
                            a[2 * h + 1] = a[2 * h + 1] + odd
                    return tuple(a)
                accs = lax.fori_loop(
                    0, hist // unroll, rbody,
                    tuple(jnp.zeros((_LANES,), jnp.float32)
                          for _ in range(cvecs)))
                orow = g * _BAGS_PER_CHUNK + b
                for c in range(cvecs):
                    out_v[orow, pl.ds(c * _LANES, _LANES)] = (
                        accs[c] * (1.0 / hist))
            return carry

        lax.fori_loop(0, chunks_per_w, chunk_body, 0)
        pltpu.sync_copy(out_v, out_hbm.at[pl.ds(wid * bags_per_w, bags_per_w)])

    return bag_mean


def _proj_body(p_ref, w_ref, b_ref, o_ref):
    o_ref[...] = lax.dot_general(
        p_ref[...], w_ref[...], (((1,), (1,)), ((), ())),
        preferred_element_type=jnp.float32) + b_ref[...]


def _project(pooled, w_proj, b_proj):
    batch, edim = pooled.shape
    odim = w_proj.shape[0]
    blk = 1024
    return pl.pallas_call(
        _proj_body,
        grid=(batch // blk,),
        in_specs=[
            pl.BlockSpec((blk, edim), lambda i: (i, 0)),
            pl.BlockSpec((odim, edim), lambda i: (0, 0)),
            pl.BlockSpec((1, odim), lambda i: (0, 0)),
        ],
        out_specs=pl.BlockSpec((blk, odim), lambda i: (i, 0)),
        out_shape=jax.ShapeDtypeStruct((batch, odim), jnp.float32),
    )(pooled, w_proj, b_proj.reshape(1, odim))


def _acc_perm(edim):
    perm = []
    for h in range(edim // 32):
        perm += [h * 32 + 2 * k for k in range(_LANES)]
        perm += [h * 32 + 2 * k + 1 for k in range(_LANES)]
    return perm


def kernel(input_, embed_weight, W_proj, b_proj):
    batch, hist = input_.shape
    num_emb, edim = embed_weight.shape
    idx_per_chunk = _BAGS_PER_CHUNK * hist
    num_chunks = batch // _BAGS_PER_CHUNK
    nseg = idx_per_chunk // _SEG
    table_p = _repack(embed_weight)
    idx3d = input_.reshape(num_chunks, nseg, _SEG)
    pooled = _make_bag_mean(batch, hist, edim, num_emb)(idx3d, table_p)
    w_perm = W_proj[:, jnp.array(_acc_perm(edim), dtype=jnp.int32)]
    return _project(pooled, w_perm, b_proj)


# speedup vs baseline: 2.0371x; 2.0371x over previous
import functools

import jax
import jax.numpy as jnp
from jax import lax
from jax.experimental import pallas as pl
from jax.experimental.pallas import tpu as pltpu
from jax.experimental.pallas import tpu_sc as plsc

_NC = 2
_NS = 16
_NW = _NC * _NS

_LANES = 16

_BAGS_PER_CHUNK = 4
_SEG = 100
_BLKP = 2048


def _repack_body(t_ref, o_ref):
    o_ref[:, 0:64] = t_ref[...].T


def _repack(embed_weight):
    num_emb, edim = embed_weight.shape
    t = embed_weight.T
    return pl.pallas_call(
        _repack_body,
        grid=(pl.cdiv(num_emb, _BLKP),),
        in_specs=[pl.BlockSpec((edim, _BLKP), lambda i: (0, i))],
        out_specs=pl.BlockSpec((_BLKP, 2 * edim), lambda i: (i, 0)),
        out_shape=jax.ShapeDtypeStruct((num_emb, 2 * edim), jnp.float32),
    )(t)


@functools.lru_cache(maxsize=None)
def _make_bag_mean(batch, hist, edim, num_emb):
    bags_per_w = batch // _NW
    chunks_per_w = bags_per_w // _BAGS_PER_CHUNK
    idx_per_chunk = _BAGS_PER_CHUNK * hist
    nseg = idx_per_chunk // _SEG
    cvecs = edim // _LANES
    wdim = 2 * edim

    mesh = plsc.VectorSubcoreMesh(core_axis_name="c", subcore_axis_name="s")

    segs_per_w = chunks_per_w * nseg

    @functools.partial(
        pl.kernel,
        out_type=jax.ShapeDtypeStruct((batch, edim), jnp.float32),
        mesh=mesh,
        scratch_types=[
            pltpu.VMEM((segs_per_w, _SEG), jnp.int32),
            pltpu.VMEM((nseg, _SEG, wdim), jnp.float32),
            pltpu.VMEM((nseg, _SEG, wdim), jnp.float32),
            pltpu.VMEM((bags_per_w, edim), jnp.float32),
            pltpu.SemaphoreType.DMA,
            pltpu.SemaphoreType.DMA,
        ],
        compiler_params=pltpu.CompilerParams(use_tc_tiling_on_sc=False),
    )
    def bag_mean(idx_hbm, table_hbm, out_hbm,
                 idx_all, rows_a, rows_b, out_v, sem_a, sem_b):
        wid = lax.axis_index("s") * _NC + lax.axis_index("c")
        pltpu.sync_copy(idx_hbm.at[wid], idx_all)

        def fire(g, rows_v, sem):
            for j in range(nseg):
                pltpu.async_copy(
                    table_hbm.at[idx_all.at[g * nseg + j]], rows_v.at[j], sem)

        def drain(g, rows_v, sem):
            for j in range(nseg):
                pltpu.make_async_copy(
                    table_hbm.at[idx_all.at[g * nseg + j]],
                    rows_v.at[j], sem).wait()

        def reduce(g, rows_v):
            for b in range(_BAGS_PER_CHUNK):
                j, r0 = divmod(b * hist, _SEG)
                unroll = 5
                def rbody(r, accs, j=j, r0=r0):
                    a = list(accs)
                    base = r0 + r * unroll
                    for u in range(unroll):
                        for c in range(cvecs):
                            a[c] = a[c] + rows_v[j, base + u,
                                                 pl.ds(c * _LANES, _LANES)]
                    return tuple(a)
                accs = lax.fori_loop(
                    0, hist // unroll, rbody,
                    tuple(jnp.zeros((_LANES,), jnp.float32)
                          for _ in range(cvecs)))
                orow = g * _BAGS_PER_CHUNK + b
                for c in range(cvecs):
                    out_v[orow, pl.ds(c * _LANES, _LANES)] = (
                        accs[c] * (1.0 / hist))

        half = chunks_per_w // 2
        fire(0, rows_a, sem_a)

        def body(k, carry):
            fire(2 * k + 1, rows_b, sem_b)
            drain(2 * k, rows_a, sem_a)
            reduce(2 * k, rows_a)

            @pl.when(k < half - 1)
            def _():
                fire(2 * k + 2, rows_a, sem_a)

            drain(2 * k + 1, rows_b, sem_b)
            reduce(2 * k + 1, rows_b)
            return carry

        lax.fori_loop(0, half, body, 0)
        pltpu.sync_copy(out_v, out_hbm.at[pl.ds(wid * bags_per_w, bags_per_w)])

    return bag_mean


def _proj_body(p_ref, w_ref, b_ref, o_ref):
    o_ref[...] = lax.dot_general(
        p_ref[...], w_ref[...], (((1,), (1,)), ((), ())),
        preferred_element_type=jnp.float32) + b_ref[...]


def _project(pooled, w_proj, b_proj):
    batch, edim = pooled.shape
    odim = w_proj.shape[0]
    blk = 1024
    return pl.pallas_call(
        _proj_body,
        grid=(batch // blk,),
        in_specs=[
            pl.BlockSpec((blk, edim), lambda i: (i, 0)),
            pl.BlockSpec((odim, edim), lambda i: (0, 0)),
            pl.BlockSpec((1, odim), lambda i: (0, 0)),
        ],
        out_specs=pl.BlockSpec((blk, odim), lambda i: (i, 0)),
        out_shape=jax.ShapeDtypeStruct((batch, odim), jnp.float32),
    )(pooled, w_proj, b_proj.reshape(1, odim))


def kernel(input_, embed_weight, W_proj, b_proj):
    batch, hist = input_.shape
    num_emb, edim = embed_weight.shape
    idx_per_chunk = _BAGS_PER_CHUNK * hist
    num_chunks = batch // _BAGS_PER_CHUNK
    nseg = idx_per_chunk // _SEG
    table_p = _repack(embed_weight)
    idx3d = input_.reshape(_NW, (batch // _NW // _BAGS_PER_CHUNK) * nseg, _SEG)
    pooled = _make_bag_mean(batch, hist, edim, num_emb)(idx3d, table_p)
    return _project(pooled, W_proj, b_proj)


# speedup vs baseline: 2.8212x; 1.3849x over previous
import functools

import jax
import jax.numpy as jnp
from jax import lax
from jax.experimental import pallas as pl
from jax.experimental.pallas import tpu as pltpu
from jax.experimental.pallas import tpu_sc as plsc

_NC = 2
_NS = 16
_NW = _NC * _NS

_LANES = 16

_BAGS_PER_CHUNK = 4
_SEG = 100
_BLKP = 8192


def _repack_body(t_ref, o_ref):
    o_ref[:, 0:64] = t_ref[...].T


def _repack(embed_weight):
    num_emb, edim = embed_weight.shape
    t = embed_weight.T
    return pl.pallas_call(
        _repack_body,
        grid=(pl.cdiv(num_emb, _BLKP),),
        in_specs=[pl.BlockSpec((edim, _BLKP), lambda i: (0, i))],
        out_specs=pl.BlockSpec((_BLKP, 2 * edim), lambda i: (i, 0)),
        out_shape=jax.ShapeDtypeStruct((num_emb, 2 * edim), jnp.float32),
    )(t)


@functools.lru_cache(maxsize=None)
def _make_bag_mean(batch, hist, edim, num_emb):
    bags_per_w = batch // _NW
    chunks_per_w = bags_per_w // _BAGS_PER_CHUNK
    idx_per_chunk = _BAGS_PER_CHUNK * hist
    nseg = idx_per_chunk // _SEG
    cvecs = edim // _LANES
    wdim = 2 * edim

    mesh = plsc.VectorSubcoreMesh(core_axis_name="c", subcore_axis_name="s")

    segs_per_w = chunks_per_w * nseg

    @functools.partial(
        pl.kernel,
        out_type=jax.ShapeDtypeStruct((batch, edim), jnp.float32),
        mesh=mesh,
        scratch_types=[
            pltpu.VMEM((segs_per_w, _SEG), jnp.int32),
            pltpu.VMEM((nseg, _SEG, wdim), jnp.float32),
            pltpu.VMEM((nseg, _SEG, wdim), jnp.float32),
            pltpu.VMEM((bags_per_w, edim), jnp.float32),
            pltpu.SemaphoreType.DMA,
            pltpu.SemaphoreType.DMA,
        ],
        compiler_params=pltpu.CompilerParams(use_tc_tiling_on_sc=False),
    )
    def bag_mean(idx_hbm, table_hbm, out_hbm,
                 idx_all, rows_a, rows_b, out_v, sem_a, sem_b):
        wid = lax.axis_index("s") * _NC + lax.axis_index("c")
        pltpu.sync_copy(idx_hbm.at[wid], idx_all)

        def fire(g, rows_v, sem):
            for j in range(nseg):
                pltpu.async_copy(
                    table_hbm.at[idx_all.at[g * nseg + j]], rows_v.at[j], sem)

        def drain(g, rows_v, sem):
            for j in range(nseg):
                pltpu.make_async_copy(
                    table_hbm.at[idx_all.at[g * nseg + j]],
                    rows_v.at[j], sem).wait()

        def reduce(g, rows_v):
            for b in range(_BAGS_PER_CHUNK):
                j, r0 = divmod(b * hist, _SEG)
                unroll = 5
                def rbody(r, accs, j=j, r0=r0):
                    a = list(accs)
                    base = r0 + r * unroll
                    for u in range(unroll):
                        for c in range(cvecs):
                            a[c] = a[c] + rows_v[j, base + u,
                                                 pl.ds(c * _LANES, _LANES)]
                    return tuple(a)
                accs = lax.fori_loop(
                    0, hist // unroll, rbody,
                    tuple(jnp.zeros((_LANES,), jnp.float32)
                          for _ in range(cvecs)))
                orow = g * _BAGS_PER_CHUNK + b
                for c in range(cvecs):
                    out_v[orow, pl.ds(c * _LANES, _LANES)] = (
                        accs[c] * (1.0 / hist))

        half = chunks_per_w // 2
        fire(0, rows_a, sem_a)

        def body(k, carry):
            fire(2 * k + 1, rows_b, sem_b)
            drain(2 * k, rows_a, sem_a)
            reduce(2 * k, rows_a)

            @pl.when(k < half - 1)
            def _():
                fire(2 * k + 2, rows_a, sem_a)

            drain(2 * k + 1, rows_b, sem_b)
            reduce(2 * k + 1, rows_b)
            return carry

        lax.fori_loop(0, half, body, 0)
        pltpu.sync_copy(out_v, out_hbm.at[pl.ds(wid * bags_per_w, bags_per_w)])

    return bag_mean


def _proj_body(p_ref, w_ref, b_ref, o_ref):
    o_ref[...] = lax.dot_general(
        p_ref[...], w_ref[...], (((1,), (1,)), ((), ())),
        preferred_element_type=jnp.float32) + b_ref[...]


def _project(pooled, w_proj, b_proj):
    batch, edim = pooled.shape
    odim = w_proj.shape[0]
    blk = 1024
    return pl.pallas_call(
        _proj_body,
        grid=(batch // blk,),
        in_specs=[
            pl.BlockSpec((blk, edim), lambda i: (i, 0)),
            pl.BlockSpec((odim, edim), lambda i: (0, 0)),
            pl.BlockSpec((1, odim), lambda i: (0, 0)),
        ],
        out_specs=pl.BlockSpec((blk, odim), lambda i: (i, 0)),
        out_shape=jax.ShapeDtypeStruct((batch, odim), jnp.float32),
    )(pooled, w_proj, b_proj.reshape(1, odim))


def kernel(input_, embed_weight, W_proj, b_proj):
    batch, hist = input_.shape
    num_emb, edim = embed_weight.shape
    idx_per_chunk = _BAGS_PER_CHUNK * hist
    num_chunks = batch // _BAGS_PER_CHUNK
    nseg = idx_per_chunk // _SEG
    table_p = _repack(embed_weight)
    idx3d = input_.reshape(_NW, (batch // _NW // _BAGS_PER_CHUNK) * nseg, _SEG)
    pooled = _make_bag_mean(batch, hist, edim, num_emb)(idx3d, table_p)
    return _project(pooled, W_proj, b_proj)


# speedup vs baseline: 2.9325x; 1.0395x over previous
import functools

import jax
import jax.numpy as jnp
from jax import lax
from jax.experimental import pallas as pl
from jax.experimental.pallas import tpu as pltpu
from jax.experimental.pallas import tpu_sc as plsc

_NC = 2
_NS = 16
_NW = _NC * _NS

_LANES = 16

_BAGS_PER_CHUNK = 4
_SEG = 100
_BLKP = 16384


def _repack_body(t_ref, o_ref):
    o_ref[:, 0:64] = t_ref[...].T


def _repack(embed_weight):
    num_emb, edim = embed_weight.shape
    t = embed_weight.T
    return pl.pallas_call(
        _repack_body,
        grid=(pl.cdiv(num_emb, _BLKP),),
        in_specs=[pl.BlockSpec((edim, _BLKP), lambda i: (0, i))],
        out_specs=pl.BlockSpec((_BLKP, 2 * edim), lambda i: (i, 0)),
        out_shape=jax.ShapeDtypeStruct((num_emb, 2 * edim), jnp.float32),
    )(t)


@functools.lru_cache(maxsize=None)
def _make_bag_mean(batch, hist, edim, num_emb):
    bags_per_w = batch // _NW
    chunks_per_w = bags_per_w // _BAGS_PER_CHUNK
    idx_per_chunk = _BAGS_PER_CHUNK * hist
    nseg = idx_per_chunk // _SEG
    cvecs = edim // _LANES
    wdim = 2 * edim

    mesh = plsc.VectorSubcoreMesh(core_axis_name="c", subcore_axis_name="s")

    segs_per_w = chunks_per_w * nseg

    @functools.partial(
        pl.kernel,
        out_type=jax.ShapeDtypeStruct((batch, edim), jnp.float32),
        mesh=mesh,
        scratch_types=[
            pltpu.VMEM((segs_per_w, _SEG), jnp.int32),
            pltpu.VMEM((nseg, _SEG, wdim), jnp.float32),
            pltpu.VMEM((nseg, _SEG, wdim), jnp.float32),
            pltpu.VMEM((bags_per_w, edim), jnp.float32),
            pltpu.SemaphoreType.DMA,
            pltpu.SemaphoreType.DMA,
        ],
        compiler_params=pltpu.CompilerParams(use_tc_tiling_on_sc=False),
    )
    def bag_mean(idx_hbm, table_hbm, out_hbm,
                 idx_all, rows_a, rows_b, out_v, sem_a, sem_b):
        wid = lax.axis_index("s") * _NC + lax.axis_index("c")
        pltpu.sync_copy(idx_hbm.at[wid], idx_all)

        def fire(g, rows_v, sem):
            for j in range(nseg):
                pltpu.async_copy(
                    table_hbm.at[idx_all.at[g * nseg + j]], rows_v.at[j], sem)

        def drain(g, rows_v, sem):
            for j in range(nseg):
                pltpu.make_async_copy(
                    table_hbm.at[idx_all.at[g * nseg + j]],
                    rows_v.at[j], sem).wait()

        def reduce(g, rows_v):
            for b in range(_BAGS_PER_CHUNK):
                j, r0 = divmod(b * hist, _SEG)
                unroll = 5
                def rbody(r, accs, j=j, r0=r0):
                    a = list(accs)
                    base = r0 + r * unroll
                    for u in range(unroll):
                        for c in range(cvecs):
                            a[c] = a[c] + rows_v[j, base + u,
                                                 pl.ds(c * _LANES, _LANES)]
                    return tuple(a)
                accs = lax.fori_loop(
                    0, hist // unroll, rbody,
                    tuple(jnp.zeros((_LANES,), jnp.float32)
                          for _ in range(cvecs)))
                orow = g * _BAGS_PER_CHUNK + b
                for c in range(cvecs):
                    out_v[orow, pl.ds(c * _LANES, _LANES)] = (
                        accs[c] * (1.0 / hist))

        half = chunks_per_w // 2
        fire(0, rows_a, sem_a)

        def body(k, carry):
            fire(2 * k + 1, rows_b, sem_b)
            drain(2 * k, rows_a, sem_a)
            reduce(2 * k, rows_a)

            @pl.when(k < half - 1)
            def _():
                fire(2 * k + 2, rows_a, sem_a)

            drain(2 * k + 1, rows_b, sem_b)
            reduce(2 * k + 1, rows_b)
            return carry

        lax.fori_loop(0, half, body, 0)
        pltpu.sync_copy(out_v, out_hbm.at[pl.ds(wid * bags_per_w, bags_per_w)])

    return bag_mean


def _proj_body(p_ref, w_ref, b_ref, o_ref):
    o_ref[...] = lax.dot_general(
        p_ref[...], w_ref[...], (((1,), (1,)), ((), ())),
        preferred_element_type=jnp.float32) + b_ref[...]


def _project(pooled, w_proj, b_proj):
    batch, edim = pooled.shape
    odim = w_proj.shape[0]
    blk = 1024
    return pl.pallas_call(
        _proj_body,
        grid=(batch // blk,),
        in_specs=[
            pl.BlockSpec((blk, edim), lambda i: (i, 0)),
            pl.BlockSpec((odim, edim), lambda i: (0, 0)),
            pl.BlockSpec((1, odim), lambda i: (0, 0)),
        ],
        out_specs=pl.BlockSpec((blk, odim), lambda i: (i, 0)),
        out_shape=jax.ShapeDtypeStruct((batch, odim), jnp.float32),
    )(pooled, w_proj, b_proj.reshape(1, odim))


def kernel(input_, embed_weight, W_proj, b_proj):
    batch, hist = input_.shape
    num_emb, edim = embed_weight.shape
    idx_per_chunk = _BAGS_PER_CHUNK * hist
    num_chunks = batch // _BAGS_PER_CHUNK
    nseg = idx_per_chunk // _SEG
    table_p = _repack(embed_weight)
    idx3d = input_.reshape(_NW, (batch // _NW // _BAGS_PER_CHUNK) * nseg, _SEG)
    pooled = _make_bag_mean(batch, hist, edim, num_emb)(idx3d, table_p)
    return _project(pooled, W_proj, b_proj)


# speedup vs baseline: 2.9669x; 1.0117x over previous
import functools

import jax
import jax.numpy as jnp
from jax import lax
from jax.experimental import pallas as pl
from jax.experimental.pallas import tpu as pltpu
from jax.experimental.pallas import tpu_sc as plsc

_NC = 2
_NS = 16
_NW = _NC * _NS

_LANES = 16

_BAGS_PER_CHUNK = 4
_SEG = 100
_BLKP = 32768


def _repack_body(t_ref, o_ref):
    o_ref[:, 0:64] = t_ref[...].T


def _repack(embed_weight):
    num_emb, edim = embed_weight.shape
    t = embed_weight.T
    return pl.pallas_call(
        _repack_body,
        grid=(pl.cdiv(num_emb, _BLKP),),
        in_specs=[pl.BlockSpec((edim, _BLKP), lambda i: (0, i))],
        out_specs=pl.BlockSpec((_BLKP, 2 * edim), lambda i: (i, 0)),
        out_shape=jax.ShapeDtypeStruct((num_emb, 2 * edim), jnp.float32),
    )(t)


@functools.lru_cache(maxsize=None)
def _make_bag_mean(batch, hist, edim, num_emb):
    bags_per_w = batch // _NW
    chunks_per_w = bags_per_w // _BAGS_PER_CHUNK
    idx_per_chunk = _BAGS_PER_CHUNK * hist
    nseg = idx_per_chunk // _SEG
    cvecs = edim // _LANES
    wdim = 2 * edim

    mesh = plsc.VectorSubcoreMesh(core_axis_name="c", subcore_axis_name="s")

    segs_per_w = chunks_per_w * nseg

    @functools.partial(
        pl.kernel,
        out_type=jax.ShapeDtypeStruct((batch, edim), jnp.float32),
        mesh=mesh,
        scratch_types=[
            pltpu.VMEM((segs_per_w, _SEG), jnp.int32),
            pltpu.VMEM((nseg, _SEG, wdim), jnp.float32),
            pltpu.VMEM((nseg, _SEG, wdim), jnp.float32),
            pltpu.VMEM((bags_per_w, edim), jnp.float32),
            pltpu.SemaphoreType.DMA,
            pltpu.SemaphoreType.DMA,
        ],
        compiler_params=pltpu.CompilerParams(use_tc_tiling_on_sc=False),
    )
    def bag_mean(idx_hbm, table_hbm, out_hbm,
                 idx_all, rows_a, rows_b, out_v, sem_a, sem_b):
        wid = lax.axis_index("s") * _NC + lax.axis_index("c")
        pltpu.sync_copy(idx_hbm.at[wid], idx_all)

        def fire(g, rows_v, sem):
            for j in range(nseg):
                pltpu.async_copy(
                    table_hbm.at[idx_all.at[g * nseg + j]], rows_v.at[j], sem)

        def drain(g, rows_v, sem):
            for j in range(nseg):
                pltpu.make_async_copy(
                    table_hbm.at[idx_all.at[g * nseg + j]],
                    rows_v.at[j], sem).wait()

        def reduce(g, rows_v):
            for b in range(_BAGS_PER_CHUNK):
                j, r0 = divmod(b * hist, _SEG)
                unroll = 5
                def rbody(r, accs, j=j, r0=r0):
                    a = list(accs)
                    base = r0 + r * unroll
                    for u in range(unroll):
                        for c in range(cvecs):
                            a[c] = a[c] + rows_v[j, base + u,
                                                 pl.ds(c * _LANES, _LANES)]
                    return tuple(a)
                accs = lax.fori_loop(
                    0, hist // unroll, rbody,
                    tuple(jnp.zeros((_LANES,), jnp.float32)
                          for _ in range(cvecs)))
                orow = g * _BAGS_PER_CHUNK + b
                for c in range(cvecs):
                    out_v[orow, pl.ds(c * _LANES, _LANES)] = (
                        accs[c] * (1.0 / hist))

        half = chunks_per_w // 2
        fire(0, rows_a, sem_a)

        def body(k, carry):
            fire(2 * k + 1, rows_b, sem_b)
            drain(2 * k, rows_a, sem_a)
            reduce(2 * k, rows_a)

            @pl.when(k < half - 1)
            def _():
                fire(2 * k + 2, rows_a, sem_a)

            drain(2 * k + 1, rows_b, sem_b)
            reduce(2 * k + 1, rows_b)
            return carry

        lax.fori_loop(0, half, body, 0)
        pltpu.sync_copy(out_v, out_hbm.at[pl.ds(wid * bags_per_w, bags_per_w)])

    return bag_mean


def _proj_body(p_ref, w_ref, b_ref, o_ref):
    o_ref[...] = lax.dot_general(
        p_ref[...], w_ref[...], (((1,), (1,)), ((), ())),
        preferred_element_type=jnp.float32) + b_ref[...]


def _project(pooled, w_proj, b_proj):
    batch, edim = pooled.shape
    odim = w_proj.shape[0]
    blk = 1024
    return pl.pallas_call(
        _proj_body,
        grid=(batch // blk,),
        in_specs=[
            pl.BlockSpec((blk, edim), lambda i: (i, 0)),
            pl.BlockSpec((odim, edim), lambda i: (0, 0)),
            pl.BlockSpec((1, odim), lambda i: (0, 0)),
        ],
        out_specs=pl.BlockSpec((blk, odim), lambda i: (i, 0)),
        out_shape=jax.ShapeDtypeStruct((batch, odim), jnp.float32),
    )(pooled, w_proj, b_proj.reshape(1, odim))


def kernel(input_, embed_weight, W_proj, b_proj):
    batch, hist = input_.shape
    num_emb, edim = embed_weight.shape
    idx_per_chunk = _BAGS_PER_CHUNK * hist
    num_chunks = batch // _BAGS_PER_CHUNK
    nseg = idx_per_chunk // _SEG
    table_p = _repack(embed_weight)
    idx3d = input_.reshape(_NW, (batch // _NW // _BAGS_PER_CHUNK) * nseg, _SEG)
    pooled = _make_bag_mean(batch, hist, edim, num_emb)(idx3d, table_p)
    return _project(pooled, W_proj, b_proj)
